# baseline (speedup 1.0000x reference)
import jax
import jax.numpy as jnp
from jax import lax
from jax.experimental import pallas as pl
from jax.experimental.pallas import tpu as pltpu
from jax.experimental.pallas import tpu_sc as plsc

V0 = 10000
V2 = 2500
B = 4
DFEAT = 128
CL1_F = 32
CL1_K = 25
CL2_F = 64
CL2_K = 25
NNZ0 = 320000
NNZ2 = 80000

V0P = 10240
V2P = 2560
NW = 32
CH = 256
FW = 128
NVJ = FW // 16


def _sc_spmm_step(VP, with_a):
    RPW = VP // NW
    PCH = 32
    mesh = plsc.VectorSubcoreMesh(core_axis_name="c", subcore_axis_name="s")
    pieces = []
    off = 0
    while off < RPW:
        pr = min(PCH, RPW - off)
        pieces.append((off, pr))
        off += pr

    def body(*refs):
        if with_a:
            (a3, koff, sub, z, rowa, cola, vala, bnds, out,
             acc, ga0, ga1, gc, colv0, colv1, rowv0, rowv1, valv0, valv1,
             bv, kv, sg0, sg1, sm0, sm1) = refs
        else:
            (sub, z, rowa, cola, vala, bnds, out,
             acc, ga0, ga1, gc, colv0, colv1, rowv0, rowv1, valv0, valv1,
             bv, kv, sg0, sg1, sm0, sm1) = refs
        ga = (ga0, ga1)
        colv = (colv0, colv1)
        rowv = (rowv0, rowv1)
        valv = (valv0, valv1)
        sg = (sg0, sg1)
        sm = (sm0, sm1)
        wid = lax.axis_index("s") * 2 + lax.axis_index("c")
        r0 = wid * RPW
        pltpu.sync_copy(bnds, bv)
        if with_a:
            pltpu.sync_copy(koff, kv)
            ko = kv[...][0]
        bvec = bv[pl.ds(wid * 16, 16)]
        c0 = bvec[0]
        s = bvec[1]
        e = bvec[2]
        nch = bvec[3]
        iota = lax.broadcasted_iota(jnp.int32, (16,), 0)

        def fire_meta(mi, p):
            c = pl.multiple_of(c0 + mi * CH, CH)
            pltpu.async_copy(cola.at[pl.ds(c, CH)], colv[p], sm[p])
            pltpu.async_copy(rowa.at[pl.ds(c, CH)], rowv[p], sm[p])
            pltpu.async_copy(vala.at[pl.ds(c, CH)], valv[p], sm[p])

        def wait_meta(p):
            pltpu.make_async_copy(cola.at[pl.ds(0, CH)], colv[p], sm[p]).wait()
            pltpu.make_async_copy(rowa.at[pl.ds(0, CH)], rowv[p], sm[p]).wait()
            pltpu.make_async_copy(vala.at[pl.ds(0, CH)], valv[p], sm[p]).wait()

        def fire_gather(p):
            pltpu.async_copy(z.at[colv[p]], ga[p], sg[p])

        def wait_gather(p):
            pltpu.make_async_copy(z.at[colv[p]], ga[p], sg[p]).wait()

        fire_meta(0, 0)
        wait_meta(0)
        fire_gather(0)
        fire_meta(1, 1)

        pltpu.sync_copy(sub.at[pl.ds(r0, RPW)], acc)
        for (po, pr) in pieces:
            if with_a:
                pltpu.sync_copy(a3.at[ko, pl.ds((r0 + po) * B, pr * B)],
                                gc.at[pl.ds(0, pr * B)])

            @plsc.parallel_loop(0, pr, unroll=2)
            def initrow(ri, po=po):
                for j in range(NVJ):
                    cur = acc[po + ri, pl.ds(j * 16, 16)]
                    if with_a:
                        avec = gc[ri * B + j // 2, pl.ds((j % 2) * 16, 16)]
                        acc[po + ri, pl.ds(j * 16, 16)] = avec - cur
                    else:
                        acc[po + ri, pl.ds(j * 16, 16)] = -cur

        def compute_chunk(p, c):
            def grp_body(g, _):
                base = g * 16
                gidx = c + base + iota
                ok = jnp.logical_and(gidx >= s, gidx < e)
                valm = jnp.where(ok, valv[p][pl.ds(base, 16)], 0.0)
                rloc = jnp.clip(rowv[p][pl.ds(base, 16)] - r0, 0, RPW - 1)
                garow = base + iota
                f0 = iota
                gabuf = ga[p]

                @plsc.parallel_loop(0, 16, unroll=4)
                def fstep(b):
                    fb = (f0 + b) & 15
                    for a in range(NVJ):
                        fvec = fb + a * 16
                        gv = plsc.load_gather(gabuf, [garow, fvec])
                        plsc.addupdate_scatter(acc, [rloc, fvec], gv * valm)

                return _

            lax.fori_loop(0, CH // 16, grp_body, None)

        def dchunk(t, _):
            for p in (0, 1):
                m = 2 * t + p
                wait_meta(1 - p)
                fire_gather(1 - p)
                wait_gather(p)
                compute_chunk(p, pl.multiple_of(c0 + m * CH, CH))
                fire_meta(m + 2, p)
            return _

        nch2 = (nch + 1) // 2
        lax.fori_loop(0, nch2, dchunk, None)

        wait_gather(0)
        wait_meta(1)

        pltpu.sync_copy(acc, out.at[pl.ds(r0, RPW)])

    scratch = [
        pltpu.VMEM((RPW, FW), jnp.float32),
        pltpu.VMEM((CH, FW), jnp.float32),
        pltpu.VMEM((CH, FW), jnp.float32),
        pltpu.VMEM((PCH * B, CL1_F), jnp.float32),
        pltpu.VMEM((CH,), jnp.int32),
        pltpu.VMEM((CH,), jnp.int32),
        pltpu.VMEM((CH,), jnp.int32),
        pltpu.VMEM((CH,), jnp.int32),
        pltpu.VMEM((CH,), jnp.float32),
        pltpu.VMEM((CH,), jnp.float32),
        pltpu.VMEM((NW * 16,), jnp.int32),
        pltpu.VMEM((16,), jnp.int32),
        pltpu.SemaphoreType.DMA,
        pltpu.SemaphoreType.DMA,
        pltpu.SemaphoreType.DMA,
        pltpu.SemaphoreType.DMA,
    ]
    return pl.kernel(
        body, mesh=mesh,
        out_type=jax.ShapeDtypeStruct((VP, FW), jnp.float32),
        scratch_types=scratch,
        compiler_params=pltpu.CompilerParams(needs_layout_passes=False,
                                             disable_bounds_checks=True),
    )


def _amm_body(lhs, rhs, out):
    x = lhs[...]
    for k in range(CL1_K):
        out[k] = jax.lax.dot_general(x, rhs[:, k * CL1_F:(k + 1) * CL1_F],
                                     (((1,), (0,)), ((), ())),
                                     preferred_element_type=jnp.float32)


def _a_matmul(x0m, c1r):
    return pl.pallas_call(
        _amm_body,
        grid=(V0P * B // 512,),
        in_specs=[pl.BlockSpec((512, DFEAT), lambda m: (m, 0)),
                  pl.BlockSpec((DFEAT, CL1_K * CL1_F), lambda m: (0, 0))],
        out_specs=pl.BlockSpec((CL1_K, 512, CL1_F), lambda m: (0, m, 0)),
        out_shape=jax.ShapeDtypeStruct((CL1_K, V0P * B, CL1_F), jnp.float32),
    )(x0m, c1r)


def _pool1_body(y, bias, out):
    t = y[...].reshape(128, 4, FW).max(axis=1)
    out[...] = jnp.maximum(t + bias[...], 0.0)


def _pool1(y1, bias_t):
    return pl.pallas_call(
        _pool1_body,
        grid=(V2P // 128,),
        in_specs=[pl.BlockSpec((512, FW), lambda m: (m, 0)),
                  pl.BlockSpec((1, FW), lambda m: (0, 0))],
        out_specs=pl.BlockSpec((128, FW), lambda m: (m, 0)),
        out_shape=jax.ShapeDtypeStruct((V2P, FW), jnp.float32),
    )(y1, bias_t)


def _comb2_body(*refs):
    xs = refs[:CL2_K]
    c2, bias, out = refs[CL2_K], refs[CL2_K + 1], refs[CL2_K + 2]
    acc = jnp.zeros((512, CL2_F), jnp.float32)
    for k in range(CL2_K):
        acc = acc + jax.lax.dot_general(xs[k][...], c2[k],
                                        (((1,), (0,)), ((), ())),
                                        preferred_element_type=jnp.float32)
    t = jnp.maximum(acc + bias[...], 0.0)
    t = t.reshape(32, 4, 4, CL2_F).max(axis=1)
    out[...] = t.reshape(128, CL2_F)


def _combine2(xs, c2, bias2):
    in_specs = ([pl.BlockSpec((512, CL1_F), lambda m: (m, 0))] * CL2_K +
                [pl.BlockSpec((CL2_K, CL1_F, CL2_F), lambda m: (0, 0, 0)),
                 pl.BlockSpec((1, CL2_F), lambda m: (0, 0))])
    return pl.pallas_call(
        _comb2_body,
        grid=(V2P * B // 512,),
        in_specs=in_specs,
        out_specs=pl.BlockSpec((128, CL2_F), lambda m: (m, 0)),
        out_shape=jax.ShapeDtypeStruct((V2P // 4 * B, CL2_F), jnp.float32),
    )(*xs, c2, bias2)


def _fc_body(h2, w1, b1, w2, b2, out, acc):
    kc = pl.program_id(0)

    @pl.when(kc == 0)
    def _():
        acc[...] = jnp.zeros_like(acc)

    lhs = h2[...].reshape(20, 4, CL2_F).transpose(1, 0, 2).reshape(4, 20 * CL2_F)
    acc[...] += jax.lax.dot_general(lhs, w1[...],
                                    (((1,), (1,)), ((), ())),
                                    preferred_element_type=jnp.float32)

    @pl.when(kc == pl.num_programs(0) - 1)
    def _():
        h1 = jnp.maximum(acc[...] + b1[...], 0.0)
        out[...] = jax.lax.dot_general(h1, w2[...],
                                       (((1,), (1,)), ((), ())),
                                       preferred_element_type=jnp.float32) + b2[...]


def _fc(h2v, w1p, b1, w2p, b2p):
    return pl.pallas_call(
        _fc_body,
        grid=(32,),
        in_specs=[pl.BlockSpec((80, CL2_F), lambda kc: (kc, 0)),
                  pl.BlockSpec((512, 1280), lambda kc: (0, kc)),
                  pl.BlockSpec((1, 512), lambda kc: (0, 0)),
                  pl.BlockSpec((16, 512), lambda kc: (0, 0)),
                  pl.BlockSpec((1, 16), lambda kc: (0, 0))],
        out_specs=pl.BlockSpec((4, 16), lambda kc: (0, 0)),
        out_shape=jax.ShapeDtypeStruct((4, 16), jnp.float32),
        scratch_shapes=[pltpu.VMEM((4, 512), jnp.float32)],
    )(h2v, w1p, b1, w2p, b2p)


def _bounds(row, VP):
    RPW = VP // NW
    edges = jnp.searchsorted(row, jnp.arange(NW + 1, dtype=jnp.int32) * RPW,
                             side="left").astype(jnp.int32)
    s = edges[:-1]
    e = edges[1:]
    c0 = (s // CH) * CH
    nch = (e - c0 + CH - 1) // CH
    nch = jnp.where(e > s, nch, 0)
    packed = jnp.stack([c0, s, e, nch], axis=1)
    return jnp.pad(packed, ((0, 0), (0, 12))).reshape(NW * 16)


def _padnnz(a):
    return jnp.pad(a, (0, 4 * CH))


def kernel(x, d, L0_row, L0_col, L0_val, L2_row, L2_col, L2_val,
           cl1_W, cl1_b, cl2_W, cl2_b, fc1_W, fc1_b, fc2_W, fc2_b):
    f32 = jnp.float32
    x0m = jnp.transpose(x, (1, 0, 2)).reshape(V0, B * DFEAT)
    x0m = jnp.pad(x0m, ((0, V0P - V0), (0, 0))).reshape(V0P * B, DFEAT)
    c1r = jnp.transpose(cl1_W.reshape(CL1_F, DFEAT, CL1_K), (1, 2, 0)
                        ).reshape(DFEAT, CL1_K * CL1_F)
    c2 = jnp.transpose(cl2_W.reshape(CL2_F, CL1_F, CL2_K), (2, 1, 0))

    row0 = _padnnz(L0_row)
    col0 = _padnnz(L0_col)
    val0 = _padnnz(L0_val)
    val0x2 = 2.0 * val0
    bnd0 = _bounds(L0_row, V0P)
    row2 = _padnnz(L2_row)
    col2 = _padnnz(L2_col)
    val2 = _padnnz(L2_val)
    val2x2 = 2.0 * val2
    bnd2 = _bounds(L2_row, V2P)

    a3 = _a_matmul(x0m, c1r)
    step1 = _sc_spmm_step(V0P, with_a=True)
    b_kp1 = a3[CL1_K - 1].reshape(V0P, FW)
    b_kp2 = jnp.zeros((V0P, FW), f32)
    for k in range(CL1_K - 2, 0, -1):
        ko = jnp.full((16,), k, jnp.int32)
        b_new = step1(a3, ko, b_kp2, b_kp1, row0, col0, val0x2, bnd0)
        b_kp2, b_kp1 = b_kp1, b_new
    ko = jnp.zeros((16,), jnp.int32)
    y1 = step1(a3, ko, b_kp2, b_kp1, row0, col0, val0, bnd0)

    h1 = _pool1(y1, jnp.tile(cl1_b, B)[None, :])

    step2 = _sc_spmm_step(V2P, with_a=False)
    zeros2 = jnp.zeros((V2P, FW), f32)
    xs = [h1]
    x_1 = step2(zeros2, h1, row2, col2, val2, bnd2)
    xs.append(x_1)
    xm2, xm1 = h1, x_1
    for k in range(2, CL2_K):
        xn = step2(xm2, xm1, row2, col2, val2x2, bnd2)
        xs.append(xn)
        xm2, xm1 = xm1, xn

    xsv = [t.reshape(V2P * B, CL1_F) for t in xs]
    h2 = _combine2(xsv, c2, cl2_b[None, :])

    w1p = jnp.pad(fc1_W, ((0, 0), (0, (640 - 625) * CL2_F)))
    w2p = jnp.pad(fc2_W, ((0, 6), (0, 0)))
    b2p = jnp.pad(fc2_b, (0, 6))
    out = _fc(h2, w1p, fc1_b[None, :], w2p, b2p[None, :])
    return out[:, :10]


# speedup vs baseline: 1.0598x; 1.0598x over previous
import jax
import jax.numpy as jnp
from jax import lax
from jax.experimental import pallas as pl
from jax.experimental.pallas import tpu as pltpu
from jax.experimental.pallas import tpu_sc as plsc

V0 = 10000
V2 = 2500
B = 4
DFEAT = 128
CL1_F = 32
CL1_K = 25
CL2_F = 64
CL2_K = 25
NNZ0 = 320000
NNZ2 = 80000

V0P = 10240
V2P = 2560
NW = 32
CH = 192
FW = 128
NVJ = FW // 16


def _sc_spmm_step(VP, with_a):
    RPW = VP // NW
    PCH = 32
    mesh = plsc.VectorSubcoreMesh(core_axis_name="c", subcore_axis_name="s")
    pieces = []
    off = 0
    while off < RPW:
        pr = min(PCH, RPW - off)
        pieces.append((off, pr))
        off += pr

    def body(*refs):
        if with_a:
            (a3, koff, sub, z, rowa, cola, vala, bnds, out,
             acc, ga0, ga1, gc, colv0, colv1, rowv0, rowv1, valv0, valv1,
             bv, kv, sg0, sg1, sm0, sm1) = refs
        else:
            (sub, z, rowa, cola, vala, bnds, out,
             acc, ga0, ga1, gc, colv0, colv1, rowv0, rowv1, valv0, valv1,
             bv, kv, sg0, sg1, sm0, sm1) = refs
        ga = (ga0, ga1)
        colv = (colv0, colv1)
        rowv = (rowv0, rowv1)
        valv = (valv0, valv1)
        sg = (sg0, sg1)
        sm = (sm0, sm1)
        wid = lax.axis_index("s") * 2 + lax.axis_index("c")
        r0 = wid * RPW
        pltpu.sync_copy(bnds, bv)
        if with_a:
            pltpu.sync_copy(koff, kv)
            ko = kv[...][0]
        bvec = bv[pl.ds(wid * 16, 16)]
        c0 = bvec[0]
        s = bvec[1]
        e = bvec[2]
        nch = bvec[3]
        iota = lax.broadcasted_iota(jnp.int32, (16,), 0)

        def fire_meta(mi, p):
            c = pl.multiple_of(c0 + mi * CH, CH)
            pltpu.async_copy(cola.at[pl.ds(c, CH)], colv[p], sm[p])
            pltpu.async_copy(rowa.at[pl.ds(c, CH)], rowv[p], sm[p])
            pltpu.async_copy(vala.at[pl.ds(c, CH)], valv[p], sm[p])

        def wait_meta(p):
            pltpu.make_async_copy(cola.at[pl.ds(0, CH)], colv[p], sm[p]).wait()
            pltpu.make_async_copy(rowa.at[pl.ds(0, CH)], rowv[p], sm[p]).wait()
            pltpu.make_async_copy(vala.at[pl.ds(0, CH)], valv[p], sm[p]).wait()

        def fire_gather(p):
            pltpu.async_copy(z.at[colv[p]], ga[p], sg[p])

        def wait_gather(p):
            pltpu.make_async_copy(z.at[colv[p]], ga[p], sg[p]).wait()

        fire_meta(0, 0)
        wait_meta(0)
        fire_gather(0)
        fire_meta(1, 1)

        pltpu.sync_copy(sub.at[pl.ds(r0, RPW)], acc)
        for (po, pr) in pieces:
            if with_a:
                pltpu.sync_copy(a3.at[ko, pl.ds((r0 + po) * B, pr * B)],
                                gc.at[pl.ds(0, pr * B)])

            @plsc.parallel_loop(0, pr, unroll=2)
            def initrow(ri, po=po):
                for j in range(NVJ):
                    cur = acc[po + ri, pl.ds(j * 16, 16)]
                    if with_a:
                        avec = gc[ri * B + j // 2, pl.ds((j % 2) * 16, 16)]
                        acc[po + ri, pl.ds(j * 16, 16)] = avec - cur
                    else:
                        acc[po + ri, pl.ds(j * 16, 16)] = -cur

        def compute_chunk(p, c):
            def grp_body(g, _):
                base = g * 16
                gidx = c + base + iota
                ok = jnp.logical_and(gidx >= s, gidx < e)
                valm = jnp.where(ok, valv[p][pl.ds(base, 16)], 0.0)
                rloc = jnp.clip(rowv[p][pl.ds(base, 16)] - r0, 0, RPW - 1)
                garow = base + iota
                f0 = iota
                gabuf = ga[p]

                @plsc.parallel_loop(0, 16, unroll=4)
                def fstep(b):
                    fb = (f0 + b) & 15
                    for a in range(NVJ):
                        fvec = fb + a * 16
                        gv = plsc.load_gather(gabuf, [garow, fvec])
                        plsc.addupdate_scatter(acc, [rloc, fvec], gv * valm)

                return _

            lax.fori_loop(0, CH // 16, grp_body, None)

        def dchunk(t, _):
            for p in (0, 1):
                m = 2 * t + p
                wait_meta(1 - p)
                fire_gather(1 - p)
                wait_gather(p)
                compute_chunk(p, pl.multiple_of(c0 + m * CH, CH))
                fire_meta(m + 2, p)
            return _

        nch2 = (nch + 1) // 2
        lax.fori_loop(0, nch2, dchunk, None)

        wait_gather(0)
        wait_meta(1)

        pltpu.sync_copy(acc, out.at[pl.ds(r0, RPW)])

    scratch = [
        pltpu.VMEM((RPW, FW), jnp.float32),
        pltpu.VMEM((CH, FW), jnp.float32),
        pltpu.VMEM((CH, FW), jnp.float32),
        pltpu.VMEM((PCH * B, CL1_F), jnp.float32),
        pltpu.VMEM((CH,), jnp.int32),
        pltpu.VMEM((CH,), jnp.int32),
        pltpu.VMEM((CH,), jnp.int32),
        pltpu.VMEM((CH,), jnp.int32),
        pltpu.VMEM((CH,), jnp.float32),
        pltpu.VMEM((CH,), jnp.float32),
        pltpu.VMEM((NW * 16,), jnp.int32),
        pltpu.VMEM((16,), jnp.int32),
        pltpu.SemaphoreType.DMA,
        pltpu.SemaphoreType.DMA,
        pltpu.SemaphoreType.DMA,
        pltpu.SemaphoreType.DMA,
    ]
    return pl.kernel(
        body, mesh=mesh,
        out_type=jax.ShapeDtypeStruct((VP, FW), jnp.float32),
        scratch_types=scratch,
        compiler_params=pltpu.CompilerParams(needs_layout_passes=False,
                                             disable_bounds_checks=True),
    )


def _amm_body(lhs, rhs, out):
    x = lhs[...]
    for k in range(CL1_K):
        out[k] = jax.lax.dot_general(x, rhs[:, k * CL1_F:(k + 1) * CL1_F],
                                     (((1,), (0,)), ((), ())),
                                     preferred_element_type=jnp.float32)


def _a_matmul(x0m, c1r):
    return pl.pallas_call(
        _amm_body,
        grid=(V0P * B // 512,),
        in_specs=[pl.BlockSpec((512, DFEAT), lambda m: (m, 0)),
                  pl.BlockSpec((DFEAT, CL1_K * CL1_F), lambda m: (0, 0))],
        out_specs=pl.BlockSpec((CL1_K, 512, CL1_F), lambda m: (0, m, 0)),
        out_shape=jax.ShapeDtypeStruct((CL1_K, V0P * B, CL1_F), jnp.float32),
    )(x0m, c1r)


def _pool1_body(y, bias, out):
    t = y[...].reshape(128, 4, FW).max(axis=1)
    out[...] = jnp.maximum(t + bias[...], 0.0)


def _pool1(y1, bias_t):
    return pl.pallas_call(
        _pool1_body,
        grid=(V2P // 128,),
        in_specs=[pl.BlockSpec((512, FW), lambda m: (m, 0)),
                  pl.BlockSpec((1, FW), lambda m: (0, 0))],
        out_specs=pl.BlockSpec((128, FW), lambda m: (m, 0)),
        out_shape=jax.ShapeDtypeStruct((V2P, FW), jnp.float32),
    )(y1, bias_t)


def _comb2_body(*refs):
    xs = refs[:CL2_K]
    c2, bias, out = refs[CL2_K], refs[CL2_K + 1], refs[CL2_K + 2]
    acc = jnp.zeros((512, CL2_F), jnp.float32)
    for k in range(CL2_K):
        acc = acc + jax.lax.dot_general(xs[k][...], c2[k],
                                        (((1,), (0,)), ((), ())),
                                        preferred_element_type=jnp.float32)
    t = jnp.maximum(acc + bias[...], 0.0)
    t = t.reshape(32, 4, 4, CL2_F).max(axis=1)
    out[...] = t.reshape(128, CL2_F)


def _combine2(xs, c2, bias2):
    in_specs = ([pl.BlockSpec((512, CL1_F), lambda m: (m, 0))] * CL2_K +
                [pl.BlockSpec((CL2_K, CL1_F, CL2_F), lambda m: (0, 0, 0)),
                 pl.BlockSpec((1, CL2_F), lambda m: (0, 0))])
    return pl.pallas_call(
        _comb2_body,
        grid=(V2P * B // 512,),
        in_specs=in_specs,
        out_specs=pl.BlockSpec((128, CL2_F), lambda m: (m, 0)),
        out_shape=jax.ShapeDtypeStruct((V2P // 4 * B, CL2_F), jnp.float32),
    )(*xs, c2, bias2)


def _fc_body(h2, w1, b1, w2, b2, out, acc):
    kc = pl.program_id(0)

    @pl.when(kc == 0)
    def _():
        acc[...] = jnp.zeros_like(acc)

    lhs = h2[...].reshape(20, 4, CL2_F).transpose(1, 0, 2).reshape(4, 20 * CL2_F)
    acc[...] += jax.lax.dot_general(lhs, w1[...],
                                    (((1,), (1,)), ((), ())),
                                    preferred_element_type=jnp.float32)

    @pl.when(kc == pl.num_programs(0) - 1)
    def _():
        h1 = jnp.maximum(acc[...] + b1[...], 0.0)
        out[...] = jax.lax.dot_general(h1, w2[...],
                                       (((1,), (1,)), ((), ())),
                                       preferred_element_type=jnp.float32) + b2[...]


def _fc(h2v, w1p, b1, w2p, b2p):
    return pl.pallas_call(
        _fc_body,
        grid=(32,),
        in_specs=[pl.BlockSpec((80, CL2_F), lambda kc: (kc, 0)),
                  pl.BlockSpec((512, 1280), lambda kc: (0, kc)),
                  pl.BlockSpec((1, 512), lambda kc: (0, 0)),
                  pl.BlockSpec((16, 512), lambda kc: (0, 0)),
                  pl.BlockSpec((1, 16), lambda kc: (0, 0))],
        out_specs=pl.BlockSpec((4, 16), lambda kc: (0, 0)),
        out_shape=jax.ShapeDtypeStruct((4, 16), jnp.float32),
        scratch_shapes=[pltpu.VMEM((4, 512), jnp.float32)],
    )(h2v, w1p, b1, w2p, b2p)


def _bounds(row, VP):
    RPW = VP // NW
    edges = jnp.searchsorted(row, jnp.arange(NW + 1, dtype=jnp.int32) * RPW,
                             side="left").astype(jnp.int32)
    s = edges[:-1]
    e = edges[1:]
    c0 = (s // CH) * CH
    nch = (e - c0 + CH - 1) // CH
    nch = jnp.where(e > s, nch, 0)
    packed = jnp.stack([c0, s, e, nch], axis=1)
    return jnp.pad(packed, ((0, 0), (0, 12))).reshape(NW * 16)


def _padnnz(a):
    return jnp.pad(a, (0, 4 * CH))


def kernel(x, d, L0_row, L0_col, L0_val, L2_row, L2_col, L2_val,
           cl1_W, cl1_b, cl2_W, cl2_b, fc1_W, fc1_b, fc2_W, fc2_b):
    f32 = jnp.float32
    x0m = jnp.transpose(x, (1, 0, 2)).reshape(V0, B * DFEAT)
    x0m = jnp.pad(x0m, ((0, V0P - V0), (0, 0))).reshape(V0P * B, DFEAT)
    c1r = jnp.transpose(cl1_W.reshape(CL1_F, DFEAT, CL1_K), (1, 2, 0)
                        ).reshape(DFEAT, CL1_K * CL1_F)
    c2 = jnp.transpose(cl2_W.reshape(CL2_F, CL1_F, CL2_K), (2, 1, 0))

    row0 = _padnnz(L0_row)
    col0 = _padnnz(L0_col)
    val0 = _padnnz(L0_val)
    val0x2 = 2.0 * val0
    bnd0 = _bounds(L0_row, V0P)
    row2 = _padnnz(L2_row)
    col2 = _padnnz(L2_col)
    val2 = _padnnz(L2_val)
    val2x2 = 2.0 * val2
    bnd2 = _bounds(L2_row, V2P)

    a3 = _a_matmul(x0m, c1r)
    step1 = _sc_spmm_step(V0P, with_a=True)
    b_kp1 = a3[CL1_K - 1].reshape(V0P, FW)
    b_kp2 = jnp.zeros((V0P, FW), f32)
    for k in range(CL1_K - 2, 0, -1):
        ko = jnp.full((16,), k, jnp.int32)
        b_new = step1(a3, ko, b_kp2, b_kp1, row0, col0, val0x2, bnd0)
        b_kp2, b_kp1 = b_kp1, b_new
    ko = jnp.zeros((16,), jnp.int32)
    y1 = step1(a3, ko, b_kp2, b_kp1, row0, col0, val0, bnd0)

    h1 = _pool1(y1, jnp.tile(cl1_b, B)[None, :])

    step2 = _sc_spmm_step(V2P, with_a=False)
    zeros2 = jnp.zeros((V2P, FW), f32)
    xs = [h1]
    x_1 = step2(zeros2, h1, row2, col2, val2, bnd2)
    xs.append(x_1)
    xm2, xm1 = h1, x_1
    for k in range(2, CL2_K):
        xn = step2(xm2, xm1, row2, col2, val2x2, bnd2)
        xs.append(xn)
        xm2, xm1 = xm1, xn

    xsv = [t.reshape(V2P * B, CL1_F) for t in xs]
    h2 = _combine2(xsv, c2, cl2_b[None, :])

    w1p = jnp.pad(fc1_W, ((0, 0), (0, (640 - 625) * CL2_F)))
    w2p = jnp.pad(fc2_W, ((0, 6), (0, 0)))
    b2p = jnp.pad(fc2_b, (0, 6))
    out = _fc(h2, w1p, fc1_b[None, :], w2p, b2p[None, :])
    return out[:, :10]
